# baseline (speedup 1.0000x reference)
import functools

import jax
import jax.numpy as jnp
from jax.experimental import pallas as pl
from jax.experimental.pallas import tpu as pltpu

_SBLK = 512


def _body(k, seq_len, s_ref, m_ref, f_ref, pools_ref, size_ref, w_ref):
    j = pl.program_id(0)
    batch = s_ref.shape[0]

    @pl.when(j == 0)
    def _compute_weights():
        s = s_ref[...]
        m = m_ref[...]
        ms = s * m
        s1 = jnp.sum(ms, axis=1, keepdims=True)
        s1s = jnp.maximum(s1, 1e-9)
        nis = (1.0 - s) * m
        s2s = jnp.maximum(jnp.sum(nis, axis=1, keepdims=True), 1e-9)
        msum = jnp.maximum(jnp.sum(m, axis=1, keepdims=True), 1e-9)

        bits = jax.lax.bitcast_convert_type(s, jnp.int32)
        keys = jnp.where(bits >= 0, bits, bits ^ jnp.int32(0x7FFFFFFF))
        lo0 = jnp.min(keys, axis=1, keepdims=True)
        hi0 = jnp.max(keys, axis=1, keepdims=True) + 1

        def _cond(lh):
            lo, hi = lh
            return jnp.any(hi - lo > 1)

        def _step(lh):
            lo, hi = lh
            mid = lo + (hi - lo) // 2
            cnt = jnp.sum((keys >= mid).astype(jnp.int32), axis=1,
                          keepdims=True)
            pred = cnt >= k
            nlo = jnp.where(pred, mid, lo)
            nhi = jnp.where(pred, hi, mid)
            exact = cnt == k
            nlo = jnp.where(exact, mid - 1, nlo)
            nhi = jnp.where(exact, mid, nhi)
            return nlo, nhi

        t, _ = jax.lax.while_loop(_cond, _step, (lo0, hi0))
        gt = keys > t
        c = jnp.sum(gt.astype(jnp.int32), axis=1, keepdims=True)
        kc = k - c
        eq = keys == t
        iota = jax.lax.broadcasted_iota(jnp.int32, s.shape, 1)

        def _step2(lh):
            lo, hi = lh
            mid = lo + (hi - lo) // 2
            cnt = jnp.sum((eq & (iota < mid)).astype(jnp.int32), axis=1,
                          keepdims=True)
            pred = cnt >= kc
            return jnp.where(pred, lo, mid), jnp.where(pred, mid, hi)

        zero = jnp.zeros((batch, 1), jnp.int32)
        full = jnp.where(kc > 0, jnp.full((batch, 1), seq_len, jnp.int32),
                         zero)
        _, u = jax.lax.while_loop(_cond, _step2, (zero, full))
        hot = gt | (eq & (iota < u))

        w_int = ms / s1s
        w_non = nis / s2s
        w_glob = m / msum
        w_hot = hot.astype(jnp.float32) * (1.0 / k)
        z = jnp.zeros_like(s)
        stacked = jnp.stack(
            [w_int, w_non, w_glob, w_hot, z, z, z, z], axis=1)
        w_ref[...] = stacked.reshape(batch * 8, seq_len)
        size_ref[...] = s1 / msum

    for b in range(4):
        wj = w_ref[b * 8:(b + 1) * 8,
                   pl.ds(pl.multiple_of(j * _SBLK, _SBLK), _SBLK)]
        part = jnp.dot(wj, f_ref[b], preferred_element_type=jnp.float32)

        @pl.when(j == 0)
        def _(b=b, part=part):
            pools_ref[b] = part

        @pl.when(j > 0)
        def _(b=b, part=part):
            pools_ref[b] += part


@jax.jit
def kernel(features, importance_scores, mask):
    batch, seq_len, hidden = features.shape
    k = max(1, int(seq_len * 0.1))
    nblk = seq_len // _SBLK

    pools, size = pl.pallas_call(
        functools.partial(_body, k, seq_len),
        grid=(nblk,),
        in_specs=[
            pl.BlockSpec((batch, seq_len), lambda j: (0, 0)),
            pl.BlockSpec((batch, seq_len), lambda j: (0, 0)),
            pl.BlockSpec((batch, _SBLK, hidden), lambda j: (0, j, 0)),
        ],
        out_specs=[
            pl.BlockSpec((batch, 8, hidden), lambda j: (0, 0, 0)),
            pl.BlockSpec((batch, 1), lambda j: (0, 0)),
        ],
        out_shape=[
            jax.ShapeDtypeStruct((batch, 8, hidden), jnp.float32),
            jax.ShapeDtypeStruct((batch, 1), jnp.float32),
        ],
        scratch_shapes=[pltpu.VMEM((batch * 8, seq_len), jnp.float32)],
    )(importance_scores, mask, features)

    return (pools[:, 0], pools[:, 1], pools[:, 3], pools[:, 2], size)


# speedup vs baseline: 1.0494x; 1.0494x over previous
import functools

import jax
import jax.numpy as jnp
from jax.experimental import pallas as pl
from jax.experimental.pallas import tpu as pltpu

_SBLK = 1024


def _body(k, seq_len, s_ref, m_ref, f_ref, pools_ref, size_ref, w_ref):
    j = pl.program_id(0)
    batch = s_ref.shape[0]

    @pl.when(j == 0)
    def _compute_weights():
        s = s_ref[...]
        m = m_ref[...]
        ms = s * m
        s1 = jnp.sum(ms, axis=1, keepdims=True)
        s1s = jnp.maximum(s1, 1e-9)
        nis = (1.0 - s) * m
        s2s = jnp.maximum(jnp.sum(nis, axis=1, keepdims=True), 1e-9)
        msum = jnp.maximum(jnp.sum(m, axis=1, keepdims=True), 1e-9)

        bits = jax.lax.bitcast_convert_type(s, jnp.int32)
        keys = jnp.where(bits >= 0, bits, bits ^ jnp.int32(0x7FFFFFFF))
        lo0 = jnp.min(keys, axis=1, keepdims=True)
        hi0 = jnp.max(keys, axis=1, keepdims=True) + 1
        n = jnp.full_like(lo0, seq_len)

        def _tofloat(key):
            b = jnp.where(key >= 0, key, key ^ jnp.int32(0x7FFFFFFF))
            return jax.lax.bitcast_convert_type(b, jnp.float32)

        def _cond(state):
            lo, hi = state[0], state[1]
            return jnp.any(hi - lo > 1)

        def _step(state):
            lo, hi, clo, chi, i = state
            bis = lo + (hi - lo) // 2
            vlo, vhi = _tofloat(lo), _tofloat(hi)
            frac = (k - clo).astype(jnp.float32) / jnp.minimum(
                (chi - clo).astype(jnp.float32), -1.0)
            vint = vlo + (vhi - vlo) * frac
            ib = jax.lax.bitcast_convert_type(vint, jnp.int32)
            interp = jnp.where(ib >= 0, ib, ib ^ jnp.int32(0x7FFFFFFF))
            mid = jnp.where(i % 2 == 0, interp, bis)
            mid = jnp.clip(mid, lo + 1, hi - 1)
            cnt = jnp.sum((keys >= mid).astype(jnp.int32), axis=1,
                          keepdims=True)
            pred = cnt >= k
            nlo = jnp.where(pred, mid, lo)
            nhi = jnp.where(pred, hi, mid)
            nclo = jnp.where(pred, cnt, clo)
            nchi = jnp.where(pred, chi, cnt)
            exact = cnt == k
            nlo = jnp.where(exact, mid - 1, nlo)
            nhi = jnp.where(exact, mid, nhi)
            return nlo, nhi, nclo, nchi, i + 1

        t, _, _, _, _ = jax.lax.while_loop(
            _cond, _step, (lo0, hi0, n, jnp.zeros_like(lo0), 0))
        gt = keys > t
        c = jnp.sum(gt.astype(jnp.int32), axis=1, keepdims=True)
        kc = k - c
        eq = keys == t
        iota = jax.lax.broadcasted_iota(jnp.int32, s.shape, 1)

        def _step2(lh):
            lo, hi = lh
            mid = lo + (hi - lo) // 2
            cnt = jnp.sum((eq & (iota < mid)).astype(jnp.int32), axis=1,
                          keepdims=True)
            pred = cnt >= kc
            return jnp.where(pred, lo, mid), jnp.where(pred, mid, hi)

        zero = jnp.zeros((batch, 1), jnp.int32)
        full = jnp.where(kc > 0, jnp.full((batch, 1), seq_len, jnp.int32),
                         zero)
        _, u = jax.lax.while_loop(_cond, _step2, (zero, full))
        hot = gt | (eq & (iota < u))

        w_int = ms / s1s
        w_non = nis / s2s
        w_glob = m / msum
        w_hot = hot.astype(jnp.float32) * (1.0 / k)
        z = jnp.zeros_like(s)
        stacked = jnp.stack(
            [w_int, w_non, w_glob, w_hot, z, z, z, z], axis=1)
        w_ref[...] = stacked.reshape(batch * 8, seq_len)
        size_ref[...] = s1 / msum

    for b in range(4):
        wj = w_ref[b * 8:(b + 1) * 8,
                   pl.ds(pl.multiple_of(j * _SBLK, _SBLK), _SBLK)]
        part = jnp.dot(wj, f_ref[b], preferred_element_type=jnp.float32)

        @pl.when(j == 0)
        def _(b=b, part=part):
            pools_ref[b] = part

        @pl.when(j > 0)
        def _(b=b, part=part):
            pools_ref[b] += part


@jax.jit
def kernel(features, importance_scores, mask):
    batch, seq_len, hidden = features.shape
    k = max(1, int(seq_len * 0.1))
    nblk = seq_len // _SBLK

    pools, size = pl.pallas_call(
        functools.partial(_body, k, seq_len),
        grid=(nblk,),
        in_specs=[
            pl.BlockSpec((batch, seq_len), lambda j: (0, 0)),
            pl.BlockSpec((batch, seq_len), lambda j: (0, 0)),
            pl.BlockSpec((batch, _SBLK, hidden), lambda j: (0, j, 0)),
        ],
        out_specs=[
            pl.BlockSpec((batch, 8, hidden), lambda j: (0, 0, 0)),
            pl.BlockSpec((batch, 1), lambda j: (0, 0)),
        ],
        out_shape=[
            jax.ShapeDtypeStruct((batch, 8, hidden), jnp.float32),
            jax.ShapeDtypeStruct((batch, 1), jnp.float32),
        ],
        scratch_shapes=[pltpu.VMEM((batch * 8, seq_len), jnp.float32)],
    )(importance_scores, mask, features)

    return (pools[:, 0], pools[:, 1], pools[:, 3], pools[:, 2], size)
